# baseline (speedup 1.0000x reference)
import jax
import jax.numpy as jnp
from jax import lax
from jax.experimental import pallas as pl
from jax.experimental.pallas import tpu as pltpu
from jax.experimental.pallas import tpu_sc as plsc

_NC, _NS = 2, 16
_NW = _NC * _NS
_BS = 128


def _sc_gather(tp, p2, n):
    s_total = p2.shape[0]
    s_per_w = s_total // _NW

    mesh = plsc.VectorSubcoreMesh(core_axis_name="c", subcore_axis_name="s")

    @pl.kernel(
        mesh=mesh,
        out_type=jax.ShapeDtypeStruct((n, 128), jnp.float32),
        scratch_types=[
            pltpu.VMEM((s_per_w, _BS), jnp.int32),
            pltpu.VMEM((_BS, 128), jnp.float32),
            pltpu.SemaphoreType.DMA,
        ],
    )
    def gather_kernel(tp_hbm, p_hbm, out_hbm, idx_v, rows_v, sem):
        wid = lax.axis_index("s") * _NC + lax.axis_index("c")
        pltpu.sync_copy(p_hbm.at[pl.ds(wid * s_per_w, s_per_w)], idx_v)
        base = wid * s_per_w * _BS

        @pl.loop(0, s_per_w)
        def _(j):
            pltpu.async_copy(tp_hbm.at[idx_v.at[j]], rows_v, sem).wait()
            pltpu.sync_copy(rows_v, out_hbm.at[pl.ds(base + j * _BS, _BS)])

    return gather_kernel(tp, p2)


def _tc_extract(packed, q, n):
    blk = 1664
    grid = n // blk

    def body(x_ref, q_ref, o_ref):
        xb = x_ref[...]
        qb = q_ref[...]
        acc = jnp.zeros((blk, 32), jnp.float32)
        for c in range(4):
            acc = acc + jnp.where(qb == c, xb[:, c * 32:(c + 1) * 32], 0.0)
        o_ref[...] = acc

    return pl.pallas_call(
        body,
        grid=(grid,),
        in_specs=[
            pl.BlockSpec((blk, 128), lambda i: (i, 0)),
            pl.BlockSpec((blk, 1), lambda i: (i, 0)),
        ],
        out_specs=pl.BlockSpec((blk, 32), lambda i: (i, 0)),
        out_shape=jax.ShapeDtypeStruct((n, 32), jnp.float32),
    )(packed, q)


def kernel(x, table):
    b, s = x.shape
    n = b * s
    d = table.shape[1]

    tp = table.reshape(table.shape[0] // 4, 4 * d)
    xf = x.reshape(n)
    p2 = (xf >> 2).reshape(n // _BS, _BS)
    q = (xf & 3).reshape(n, 1)

    packed = _sc_gather(tp, p2, n)
    out = _tc_extract(packed, q, n)
    return out.reshape(b, s, d)


# speedup vs baseline: 1.0151x; 1.0151x over previous
import jax
import jax.numpy as jnp
from jax import lax
from jax.experimental import pallas as pl
from jax.experimental.pallas import tpu as pltpu
from jax.experimental.pallas import tpu_sc as plsc

_NC, _NS = 2, 16
_NW = _NC * _NS
_BS = 128


def _sc_gather(tp, p2, n):
    s_total = p2.shape[0]
    s_per_w = s_total // _NW

    mesh = plsc.VectorSubcoreMesh(core_axis_name="c", subcore_axis_name="s")

    @pl.kernel(
        mesh=mesh,
        out_type=jax.ShapeDtypeStruct((n, 128), jnp.float32),
        scratch_types=[
            pltpu.VMEM((s_per_w, _BS), jnp.int32),
            pltpu.VMEM((_BS, 128), jnp.float32),
            pltpu.VMEM((_BS, 128), jnp.float32),
            pltpu.VMEM((_BS, 128), jnp.float32),
            pltpu.VMEM((_BS, 128), jnp.float32),
            pltpu.SemaphoreType.DMA,
            pltpu.SemaphoreType.DMA,
            pltpu.SemaphoreType.DMA,
            pltpu.SemaphoreType.DMA,
            pltpu.SemaphoreType.DMA,
            pltpu.SemaphoreType.DMA,
            pltpu.SemaphoreType.DMA,
            pltpu.SemaphoreType.DMA,
        ],
    )
    def gather_kernel(tp_hbm, p_hbm, out_hbm, idx_v,
                      b0, b1, b2, b3,
                      g0, g1, g2, g3, w0, w1, w2, w3):
        wid = lax.axis_index("s") * _NC + lax.axis_index("c")
        pltpu.sync_copy(p_hbm.at[pl.ds(wid * s_per_w, s_per_w)], idx_v)
        base = wid * s_per_w * _BS

        bufs = (b0, b1, b2, b3)
        gsems = (g0, g1, g2, g3)
        wsems = (w0, w1, w2, w3)

        def start_g(i, t):
            pltpu.make_async_copy(
                tp_hbm.at[idx_v.at[i]], bufs[t], gsems[t]
            ).start()

        def wait_g(t):
            pltpu.make_async_copy(tp_hbm.at[idx_v.at[0]], bufs[t],
                                  gsems[t]).wait()

        def start_w(i, t):
            pltpu.make_async_copy(
                bufs[t], out_hbm.at[pl.ds(base + i * _BS, _BS)], wsems[t]
            ).start()

        def wait_w(t):
            pltpu.make_async_copy(bufs[t],
                                  out_hbm.at[pl.ds(base, _BS)],
                                  wsems[t]).wait()

        start_g(0, 0)
        start_g(1, 1)
        start_g(2, 2)
        wait_g(0); start_w(0, 0); start_g(3, 3)
        wait_g(1); start_w(1, 1); wait_w(0); start_g(4, 0)
        wait_g(2); start_w(2, 2); wait_w(1); start_g(5, 1)
        wait_g(3); start_w(3, 3); wait_w(2); start_g(6, 2)

        @pl.loop(4, s_per_w, step=4)
        def _(j):
            for t in range(4):
                i = j + t
                wait_g(t)
                start_w(i, t)
                nxt = (t + 3) % 4
                wait_w(nxt)

                @pl.when(i + 3 < s_per_w)
                def _():
                    start_g(i + 3, nxt)

        wait_w(3)

    return gather_kernel(tp, p2)


def _tc_extract(packed, q, n):
    blk = 1664
    grid = n // blk

    def body(x_ref, q_ref, o_ref):
        xb = x_ref[...]
        qb = q_ref[...]
        acc = jnp.zeros((blk, 32), jnp.float32)
        for c in range(4):
            acc = acc + jnp.where(qb == c, xb[:, c * 32:(c + 1) * 32], 0.0)
        o_ref[...] = acc

    return pl.pallas_call(
        body,
        grid=(grid,),
        in_specs=[
            pl.BlockSpec((blk, 128), lambda i: (i, 0)),
            pl.BlockSpec((blk, 1), lambda i: (i, 0)),
        ],
        out_specs=pl.BlockSpec((blk, 32), lambda i: (i, 0)),
        out_shape=jax.ShapeDtypeStruct((n, 32), jnp.float32),
    )(packed, q)


def kernel(x, table):
    b, s = x.shape
    n = b * s
    d = table.shape[1]

    tp = table.reshape(table.shape[0] // 4, 4 * d)
    xf = x.reshape(n)
    p2 = (xf >> 2).reshape(n // _BS, _BS)
    q = (xf & 3).reshape(n, 1)

    packed = _sc_gather(tp, p2, n)
    out = _tc_extract(packed, q, n)
    return out.reshape(b, s, d)


# speedup vs baseline: 1.5512x; 1.5281x over previous
import jax
import jax.numpy as jnp
from jax import lax
from jax.experimental import pallas as pl
from jax.experimental.pallas import tpu as pltpu
from jax.experimental.pallas import tpu_sc as plsc

_NC, _NS = 2, 16
_NW = _NC * _NS
_BS = 128
_D = 32


def _sc_gather(tp, p2, q2, n):
    s_total = p2.shape[0]
    s_per_w = s_total // _NW

    mesh = plsc.VectorSubcoreMesh(core_axis_name="c", subcore_axis_name="s")

    @pl.kernel(
        mesh=mesh,
        out_type=jax.ShapeDtypeStruct((n, _D), jnp.float32),
        scratch_types=[
            pltpu.VMEM((s_per_w, _BS), jnp.int32),
            pltpu.VMEM((s_per_w, _BS), jnp.int32),
            pltpu.VMEM((_BS, _BS), jnp.float32),
            pltpu.VMEM((_BS, _BS), jnp.float32),
            pltpu.VMEM((_BS, _BS), jnp.float32),
            pltpu.VMEM((_BS, _D), jnp.float32),
            pltpu.VMEM((_BS, _D), jnp.float32),
            pltpu.VMEM((_BS, _D), jnp.float32),
            pltpu.SemaphoreType.DMA,
            pltpu.SemaphoreType.DMA,
            pltpu.SemaphoreType.DMA,
            pltpu.SemaphoreType.DMA,
            pltpu.SemaphoreType.DMA,
            pltpu.SemaphoreType.DMA,
        ],
    )
    def gather_kernel(tp_hbm, p_hbm, q_hbm, out_hbm, idx_v, qoff_v,
                      b0, b1, b2, o0, o1, o2,
                      g0, g1, g2, w0, w1, w2):
        wid = lax.axis_index("s") * _NC + lax.axis_index("c")
        pltpu.sync_copy(p_hbm.at[pl.ds(wid * s_per_w, s_per_w)], idx_v)
        pltpu.sync_copy(q_hbm.at[pl.ds(wid * s_per_w, s_per_w)], qoff_v)
        base = wid * s_per_w * _BS

        bufs = (b0, b1, b2)
        obufs = (o0, o1, o2)
        gsems = (g0, g1, g2)
        wsems = (w0, w1, w2)
        iota = lax.iota(jnp.int32, 16)

        def start_g(i, t):
            pltpu.make_async_copy(
                tp_hbm.at[idx_v.at[i]], bufs[t], gsems[t]
            ).start()

        def wait_g(t):
            pltpu.make_async_copy(tp_hbm.at[idx_v.at[0]], bufs[t],
                                  gsems[t]).wait()

        def start_w(i, t):
            pltpu.make_async_copy(
                obufs[t], out_hbm.at[pl.ds(base + i * _BS, _BS)], wsems[t]
            ).start()

        def wait_w(t):
            pltpu.make_async_copy(obufs[t],
                                  out_hbm.at[pl.ds(base, _BS)],
                                  wsems[t]).wait()

        def extract(i, t):
            for g in range(_BS // 16):
                qvec = qoff_v[i, pl.ds(g * 16, 16)]
                for k in range(16):
                    r = g * 16 + k
                    qoff = qvec[k]
                    for h in range(_D // 16):
                        obufs[t][r, pl.ds(h * 16, 16)] = (
                            bufs[t][r, pl.ds(qoff + h * 16, 16)])

        for t in range(3):
            start_g(t, t)

        n_loop = (s_per_w // 3) * 3

        @pl.loop(0, n_loop, step=3)
        def _(j):
            for t in range(3):
                i = j + t
                wait_g(t)

                @pl.when(i >= 3)
                def _():
                    wait_w(t)

                extract(i, t)
                start_w(i, t)

                @pl.when(i + 3 < s_per_w)
                def _():
                    start_g(i + 3, t)

        for i in range(n_loop, s_per_w):
            t = i % 3
            wait_g(t)
            wait_w(t)
            extract(i, t)
            start_w(i, t)
        for t in range(3):
            wait_w(t)

    return gather_kernel(tp, p2, q2)


def kernel(x, table):
    b, s = x.shape
    n = b * s
    d = table.shape[1]

    tp = table.reshape(table.shape[0] // 4, 4 * d)
    xf = x.reshape(n // _BS, _BS)
    p2 = xf >> 2
    q2 = (xf & 3) * d

    out = _sc_gather(tp, p2, q2, n)
    return out.reshape(b, s, d)


# speedup vs baseline: 1.7837x; 1.1499x over previous
import jax
import jax.numpy as jnp
from jax import lax
from jax.experimental import pallas as pl
from jax.experimental.pallas import tpu as pltpu
from jax.experimental.pallas import tpu_sc as plsc

_NC, _NS = 2, 16
_NW = _NC * _NS
_BS = 104
_D = 32


def _sc_gather(tp, p2, q2, b, s):
    s_total = p2.shape[0]
    s_per_w = s_total // _NW
    sent_per_stream = _BS // 26

    mesh = plsc.VectorSubcoreMesh(core_axis_name="c", subcore_axis_name="s")

    @pl.kernel(
        mesh=mesh,
        out_type=jax.ShapeDtypeStruct((b, s, _D), jnp.float32),
        scratch_types=[
            pltpu.VMEM((s_per_w, _BS), jnp.int32),
            pltpu.VMEM((s_per_w, _BS), jnp.int32),
            pltpu.VMEM((_BS, 4 * _D), jnp.float32),
            pltpu.VMEM((_BS, 4 * _D), jnp.float32),
            pltpu.VMEM((_BS, 4 * _D), jnp.float32),
            pltpu.VMEM((sent_per_stream, 26, _D), jnp.float32),
            pltpu.VMEM((sent_per_stream, 26, _D), jnp.float32),
            pltpu.VMEM((sent_per_stream, 26, _D), jnp.float32),
            pltpu.SemaphoreType.DMA,
            pltpu.SemaphoreType.DMA,
            pltpu.SemaphoreType.DMA,
            pltpu.SemaphoreType.DMA,
            pltpu.SemaphoreType.DMA,
            pltpu.SemaphoreType.DMA,
        ],
    )
    def gather_kernel(tp_hbm, p_hbm, q_hbm, out_hbm, idx_v, qoff_v,
                      b0, b1, b2, o0, o1, o2,
                      g0, g1, g2, w0, w1, w2):
        wid = lax.axis_index("s") * _NC + lax.axis_index("c")
        pltpu.sync_copy(p_hbm.at[pl.ds(wid * s_per_w, s_per_w)], idx_v)
        pltpu.sync_copy(q_hbm.at[pl.ds(wid * s_per_w, s_per_w)], qoff_v)
        sent_base = wid * s_per_w * sent_per_stream

        bufs = (b0, b1, b2)
        obufs = (o0, o1, o2)
        gsems = (g0, g1, g2)
        wsems = (w0, w1, w2)

        def start_g(i, t):
            pltpu.make_async_copy(
                tp_hbm.at[idx_v.at[i]], bufs[t], gsems[t]
            ).start()

        def wait_g(t):
            pltpu.make_async_copy(tp_hbm.at[idx_v.at[0]], bufs[t],
                                  gsems[t]).wait()

        def start_w(i, t):
            dst = out_hbm.at[
                pl.ds(sent_base + i * sent_per_stream, sent_per_stream)]
            pltpu.make_async_copy(obufs[t], dst, wsems[t]).start()

        def wait_w(t):
            pltpu.make_async_copy(
                obufs[t], out_hbm.at[pl.ds(0, sent_per_stream)], wsems[t]
            ).wait()

        def extract(i, t):
            for g in range(7):
                lo = 0 if g < 6 else 8
                gbase = g * 16 if g < 6 else 88
                qvec = qoff_v[i, pl.ds(gbase, 16)]
                for k in range(lo, 16):
                    r = gbase + k
                    qoff = qvec[k]
                    for h in range(_D // 16):
                        obufs[t][r // 26, r % 26, pl.ds(h * 16, 16)] = (
                            bufs[t][r, pl.ds(qoff + h * 16, 16)])

        for t in range(3):
            start_g(t, t)

        n_loop = (s_per_w // 3) * 3

        @pl.loop(0, n_loop, step=3)
        def _(j):
            for t in range(3):
                i = j + t
                wait_g(t)

                @pl.when(i >= 3)
                def _():
                    wait_w(t)

                extract(i, t)
                start_w(i, t)

                @pl.when(i + 3 < s_per_w)
                def _():
                    start_g(i + 3, t)

        for i in range(n_loop, s_per_w):
            t = i % 3
            wait_g(t)
            wait_w(t)
            extract(i, t)
            start_w(i, t)
        for t in range(3):
            wait_w(t)

    return gather_kernel(tp, p2, q2)


def kernel(x, table):
    b, s = x.shape
    n = b * s
    d = table.shape[1]

    tp = table.reshape(table.shape[0] // 4, 4 * d)
    xf = x.reshape(n // _BS, _BS)
    p2 = xf >> 2
    q2 = (xf & 3) * d

    return _sc_gather(tp, p2, q2, b, s)


# speedup vs baseline: 1.7854x; 1.0010x over previous
import jax
import jax.numpy as jnp
from jax import lax
from jax.experimental import pallas as pl
from jax.experimental.pallas import tpu as pltpu
from jax.experimental.pallas import tpu_sc as plsc

_NC, _NS = 2, 16
_NW = _NC * _NS
_BS = 104
_D = 32


def _sc_gather(tp, xf, b, s):
    s_total = xf.shape[0] // _BS
    s_per_w = s_total // _NW
    sent_per_stream = _BS // 26

    mesh = plsc.VectorSubcoreMesh(core_axis_name="c", subcore_axis_name="s")

    @pl.kernel(
        mesh=mesh,
        out_type=jax.ShapeDtypeStruct((b, s, _D), jnp.float32),
        scratch_types=[
            pltpu.VMEM((s_per_w * _BS,), jnp.int32),
            pltpu.VMEM((s_per_w * _BS,), jnp.int32),
            pltpu.VMEM((_BS, 4 * _D), jnp.float32),
            pltpu.VMEM((_BS, 4 * _D), jnp.float32),
            pltpu.VMEM((_BS, 4 * _D), jnp.float32),
            pltpu.VMEM((sent_per_stream, 26, _D), jnp.float32),
            pltpu.VMEM((sent_per_stream, 26, _D), jnp.float32),
            pltpu.VMEM((sent_per_stream, 26, _D), jnp.float32),
            pltpu.SemaphoreType.DMA,
            pltpu.SemaphoreType.DMA,
            pltpu.SemaphoreType.DMA,
            pltpu.SemaphoreType.DMA,
            pltpu.SemaphoreType.DMA,
            pltpu.SemaphoreType.DMA,
        ],
    )
    def gather_kernel(tp_hbm, x_hbm, out_hbm, idx_v, qoff_v,
                      b0, b1, b2, o0, o1, o2,
                      g0, g1, g2, w0, w1, w2):
        wid = lax.axis_index("s") * _NC + lax.axis_index("c")
        n_w = s_per_w * _BS
        pltpu.sync_copy(x_hbm.at[pl.ds(wid * n_w, n_w)], idx_v)
        sent_base = wid * s_per_w * sent_per_stream

        @pl.loop(0, n_w, step=16)
        def _(j):
            v = idx_v[pl.ds(j, 16)]
            qoff_v[pl.ds(j, 16)] = (v & 3) * _D
            idx_v[pl.ds(j, 16)] = v >> 2

        bufs = (b0, b1, b2)
        obufs = (o0, o1, o2)
        gsems = (g0, g1, g2)
        wsems = (w0, w1, w2)

        def start_g(i, t):
            pltpu.make_async_copy(
                tp_hbm.at[idx_v.at[pl.ds(i * _BS, _BS)]], bufs[t], gsems[t]
            ).start()

        def wait_g(t):
            pltpu.make_async_copy(tp_hbm.at[idx_v.at[pl.ds(0, _BS)]], bufs[t],
                                  gsems[t]).wait()

        def start_w(i, t):
            dst = out_hbm.at[
                pl.ds(sent_base + i * sent_per_stream, sent_per_stream)]
            pltpu.make_async_copy(obufs[t], dst, wsems[t]).start()

        def wait_w(t):
            pltpu.make_async_copy(
                obufs[t], out_hbm.at[pl.ds(0, sent_per_stream)], wsems[t]
            ).wait()

        def extract(i, t):
            for g in range(7):
                lo = 0 if g < 6 else 8
                gbase = g * 16 if g < 6 else 88
                qvec = qoff_v[pl.ds(i * _BS + gbase, 16)]
                for k in range(lo, 16):
                    r = gbase + k
                    qoff = qvec[k]
                    for h in range(_D // 16):
                        obufs[t][r // 26, r % 26, pl.ds(h * 16, 16)] = (
                            bufs[t][r, pl.ds(qoff + h * 16, 16)])

        for t in range(3):
            start_g(t, t)

        n_loop = (s_per_w // 3) * 3

        @pl.loop(0, n_loop, step=3)
        def _(j):
            for t in range(3):
                i = j + t
                wait_g(t)

                @pl.when(i >= 3)
                def _():
                    wait_w(t)

                extract(i, t)
                start_w(i, t)

                @pl.when(i + 3 < s_per_w)
                def _():
                    start_g(i + 3, t)

        for i in range(n_loop, s_per_w):
            t = i % 3
            wait_g(t)
            wait_w(t)
            extract(i, t)
            start_w(i, t)
        for t in range(3):
            wait_w(t)

    return gather_kernel(tp, xf)


def kernel(x, table):
    b, s = x.shape
    n = b * s
    d = table.shape[1]

    tp = table.reshape(table.shape[0] // 4, 4 * d)
    return _sc_gather(tp, x.reshape(n), b, s)
